# baseline (speedup 1.0000x reference)
import functools
import math

import jax
import jax.numpy as jnp
from jax import lax
from jax.experimental import pallas as pl
from jax.experimental.pallas import tpu as pltpu
from jax.experimental.pallas import tpu_sc as plsc

N0 = 10000
NP0 = 10240
E0 = 320000
D = 128
DEPTH = 3
NW = 32
F32 = jnp.float32

LVL = {1: (5000, 5120), 2: (2500, 2560), 3: (1250, 1280)}


NPH = NP0 // 2
NACC = NPH + 128


def _spmm_sc_body(y_hbm, uu_hbm, vv_hbm, zeros_hbm, out_hbm,
                  ubuf, vbuf, rows, acc_sh, sem):
    C = ubuf.shape[0]
    cid = lax.axis_index("c")
    sid = lax.axis_index("s")
    rpt = NACC // 16
    pltpu.sync_copy(zeros_hbm.at[pl.ds(sid * rpt, rpt)],
                    acc_sh.at[pl.ds(sid * rpt, rpt)])
    plsc.subcore_barrier()
    ep_total = vv_hbm.shape[0]
    epw = ep_total // 16
    nch = epw // C

    def chunk(t, carry):
        off = sid * epw + t * C
        pltpu.sync_copy(uu_hbm.at[cid].at[pl.ds(off, C)], ubuf)
        pltpu.sync_copy(vv_hbm.at[pl.ds(off, C)], vbuf)
        pltpu.async_copy(y_hbm.at[vbuf], rows, sem).wait()
        pltpu.sync_copy(rows, acc_sh.at[ubuf], add=True)
        return carry

    lax.fori_loop(0, nch, chunk, 0)
    plsc.subcore_barrier()
    dpt = NPH // 16
    pltpu.sync_copy(acc_sh.at[pl.ds(sid * dpt, dpt)],
                    out_hbm.at[cid].at[pl.ds(sid * dpt, dpt)])


def _spmm_sc(y, uu2, vv):
    C = 512
    zeros = jnp.zeros((NACC, D), F32)
    mesh = plsc.VectorSubcoreMesh(core_axis_name="c", subcore_axis_name="s")
    f = pl.kernel(
        _spmm_sc_body,
        out_type=jax.ShapeDtypeStruct((2, NPH, D), F32),
        mesh=mesh,
        scratch_types=[
            pltpu.VMEM((C,), jnp.int32),
            pltpu.VMEM((C,), jnp.int32),
            pltpu.VMEM((C, D), F32),
            pltpu.VMEM_SHARED((NACC, D), F32),
            pltpu.SemaphoreType.DMA,
        ],
    )
    return f(y, uu2, vv, zeros)


def _gather_rows_body(tab_hbm, idx_hbm, out_hbm, ibuf, rows, sem, *, C):
    cid = lax.axis_index("c")
    sid = lax.axis_index("s")
    wid = sid * 2 + cid
    B = idx_hbm.shape[0]
    bpw = B // NW
    nch = bpw // C

    def chunk(t, carry):
        off = wid * bpw + t * C
        pltpu.sync_copy(idx_hbm.at[pl.ds(off, C)], ibuf)
        pltpu.async_copy(tab_hbm.at[ibuf], rows, sem).wait()
        pltpu.sync_copy(rows, out_hbm.at[pl.ds(off, C)])
        return carry

    lax.fori_loop(0, nch, chunk, 0)


def _gather_rows(tab, idx, C):
    B = idx.shape[0]
    W = tab.shape[1]
    mesh = plsc.VectorSubcoreMesh(core_axis_name="c", subcore_axis_name="s")
    f = pl.kernel(
        functools.partial(_gather_rows_body, C=C),
        out_type=jax.ShapeDtypeStruct((B, W), tab.dtype),
        mesh=mesh,
        scratch_types=[
            pltpu.VMEM((C,), jnp.int32),
            pltpu.VMEM((C, W), tab.dtype),
            pltpu.SemaphoreType.DMA,
        ],
    )
    return f(tab, idx)


def _onehot_body(perm_ref, o_ref, *, bm, bn):
    i = pl.program_id(0)
    j = pl.program_id(1)
    rows = i * bm + lax.broadcasted_iota(jnp.int32, (bm, bn), 0)
    pv = perm_ref[0, 0, :]
    o_ref[...] = (rows == pv[None, :]).astype(jnp.bfloat16)


def _onehot(perm_p, np_rows, bm=256, bn=256):
    kp = perm_p.shape[0]
    p3 = perm_p.reshape(1, 1, kp)
    return pl.pallas_call(
        functools.partial(_onehot_body, bm=bm, bn=bn),
        grid=(np_rows // bm, kp // bn),
        in_specs=[pl.BlockSpec((1, 1, bn), lambda i, j: (0, 0, j))],
        out_specs=pl.BlockSpec((bm, bn), lambda i, j: (i, j)),
        out_shape=jax.ShapeDtypeStruct((np_rows, kp), jnp.bfloat16),
    )(p3)


def _mm_body(a_ref, b_ref, o_ref, acc, *, nk):
    kk = pl.program_id(2)

    @pl.when(kk == 0)
    def _():
        acc[...] = jnp.zeros_like(acc)

    a = a_ref[...].astype(jnp.bfloat16)
    acc[...] += jnp.dot(a, b_ref[...], preferred_element_type=F32)

    @pl.when(kk == nk - 1)
    def _():
        o_ref[...] = acc[...].astype(jnp.bfloat16)


def _mm_sel(A, S, bm=1280, bn=1280, bk=512):
    m, K = A.shape
    n = S.shape[1]
    bm = min(bm, m)
    bn = min(bn, n)
    nk = K // bk
    return pl.pallas_call(
        functools.partial(_mm_body, nk=nk),
        grid=(m // bm, n // bn, nk),
        in_specs=[
            pl.BlockSpec((bm, bk), lambda i, j, k: (i, k)),
            pl.BlockSpec((bk, bn), lambda i, j, k: (k, j)),
        ],
        out_specs=pl.BlockSpec((bm, bn), lambda i, j, k: (i, j)),
        out_shape=jax.ShapeDtypeStruct((m, n), jnp.bfloat16),
        scratch_shapes=[pltpu.VMEM((bm, bn), F32)],
    )(A, S)


def _zw_body(x_ref, w_ref, deg_ref, b_ref, add_ref, o_ref, *, use_add, relu):
    x = x_ref[...]
    if use_add:
        x = x + add_ref[...]
    acc = jnp.dot(x, w_ref[...], preferred_element_type=F32)
    deg = deg_ref[...]
    rs = jnp.where(deg > 0.0, lax.rsqrt(deg), 0.0)
    out = acc * rs[:, None] + b_ref[...][None, :]
    if relu:
        out = jnp.maximum(out, 0.0)
    o_ref[...] = out


def _zw(x, w, deg, b, add=None, relu=False, bm=256):
    n = x.shape[0]
    use_add = add is not None
    if add is None:
        add = x
    grid = (n // bm,)
    return pl.pallas_call(
        functools.partial(_zw_body, use_add=use_add, relu=relu),
        grid=grid,
        in_specs=[
            pl.BlockSpec((bm, D), lambda i: (i, 0)),
            pl.BlockSpec((D, D), lambda i: (0, 0)),
            pl.BlockSpec((bm,), lambda i: (i,)),
            pl.BlockSpec((D,), lambda i: (0,)),
            pl.BlockSpec((bm, D), lambda i: (i, 0)),
        ],
        out_specs=pl.BlockSpec((bm, D), lambda i: (i, 0)),
        out_shape=jax.ShapeDtypeStruct((n, D), F32),
    )(x, w, deg, b, add)


def _combine_body(s0_ref, z_ref, deg_ref, b_ref, o_ref,
                  *, coef, relu, nvalid, bm):
    i = pl.program_id(0)
    s = s0_ref[...] + coef * z_ref[...]
    deg = deg_ref[...]
    rs = jnp.where(deg > 0.0, lax.rsqrt(deg), 0.0)
    out = s * rs[:, None] + b_ref[...][None, :]
    if relu:
        out = jnp.maximum(out, 0.0)
    rows = i * bm + lax.broadcasted_iota(jnp.int32, (bm, 1), 0)
    o_ref[...] = jnp.where(rows < nvalid, out, 0.0)


def _combine(s0, z, deg, b, coef, relu, nvalid, bm=512):
    n = z.shape[0]
    return pl.pallas_call(
        functools.partial(_combine_body, coef=coef, relu=relu,
                          nvalid=nvalid, bm=bm),
        grid=(n // bm,),
        in_specs=[
            pl.BlockSpec((bm, D), lambda i: (i, 0)),
            pl.BlockSpec((bm, D), lambda i: (i, 0)),
            pl.BlockSpec((bm,), lambda i: (i,)),
            pl.BlockSpec((D,), lambda i: (0,)),
        ],
        out_specs=pl.BlockSpec((bm, D), lambda i: (i, 0)),
        out_shape=jax.ShapeDtypeStruct((n, D), F32),
    )(s0, z, deg, b)


def _dense_gcn_body(m_ref, z_ref, zrow_ref, deg_ref, b_ref, o_ref, acc,
                    *, nk, relu, nvalid, bm):
    kk = pl.program_id(1)

    @pl.when(kk == 0)
    def _():
        acc[...] = jnp.zeros_like(acc)

    acc[...] += jnp.dot(m_ref[...], z_ref[...], preferred_element_type=F32)
    i = pl.program_id(0)

    @pl.when(kk == nk - 1)
    def _():
        deg = deg_ref[...]
        rs = jnp.where(deg > 0.0, lax.rsqrt(deg), 0.0)
        out = rs[:, None] * (acc[...] + zrow_ref[...]) + b_ref[...][None, :]
        if relu:
            out = jnp.maximum(out, 0.0)
        rows = i * bm + lax.broadcasted_iota(jnp.int32, (bm, 1), 0)
        o_ref[...] = jnp.where(rows < nvalid, out, 0.0)


def _dense_gcn(M, Z, deg, b, relu, nvalid, bm=256, bk=640):
    n = M.shape[0]
    nk = n // bk
    return pl.pallas_call(
        functools.partial(_dense_gcn_body, nk=nk, relu=relu,
                          nvalid=nvalid, bm=bm),
        grid=(n // bm, nk),
        in_specs=[
            pl.BlockSpec((bm, bk), lambda i, k: (i, k)),
            pl.BlockSpec((bk, D), lambda i, k: (k, 0)),
            pl.BlockSpec((bm, D), lambda i, k: (i, 0)),
            pl.BlockSpec((bm,), lambda i, k: (i,)),
            pl.BlockSpec((D,), lambda i, k: (0,)),
        ],
        out_specs=pl.BlockSpec((bm, D), lambda i, k: (i, 0)),
        out_shape=jax.ShapeDtypeStruct((n, D), F32),
        scratch_shapes=[pltpu.VMEM((bm, D), F32)],
    )(M, Z, Z, deg, b)


def _bigmm_body(l_ref, r_ref, g_ref, d_ref, acc, *, nk, nj, nvalid, bm, bn):
    i = pl.program_id(0)
    j = pl.program_id(1)
    kk = pl.program_id(2)

    @pl.when(kk == 0)
    def _():
        acc[...] = jnp.zeros_like(acc)

    a = l_ref[...].astype(jnp.bfloat16)
    b = r_ref[...].astype(jnp.bfloat16)
    acc[...] += jnp.dot(a, b, preferred_element_type=F32)

    @pl.when(kk == nk - 1)
    def _():
        rows = i * bm + lax.broadcasted_iota(jnp.int32, (bm, bn), 0)
        cols = j * bn + lax.broadcasted_iota(jnp.int32, (bm, bn), 1)
        valid = (rows < nvalid) & (cols < nvalid)
        g = jnp.where(rows == cols, 1.0, acc[...])
        g = jnp.where(valid, g, 0.0)
        g_ref[...] = g
        part = jnp.sum(g, axis=1).reshape(1, 1, bm)

        @pl.when(j == 0)
        def _():
            d_ref[...] = part

        @pl.when(j > 0)
        def _():
            d_ref[...] += part


def _bigmm(L, R, nvalid, bm=1280, bn=1280, bk=512):
    m, K = L.shape
    n = R.shape[1]
    bm = min(bm, m)
    bn = min(bn, n)
    nk = K // bk
    nj = n // bn
    return pl.pallas_call(
        functools.partial(_bigmm_body, nk=nk, nj=nj, nvalid=nvalid,
                          bm=bm, bn=bn),
        grid=(m // bm, nj, nk),
        in_specs=[
            pl.BlockSpec((bm, bk), lambda i, j, k: (i, k)),
            pl.BlockSpec((bk, bn), lambda i, j, k: (k, j)),
        ],
        out_specs=[
            pl.BlockSpec((bm, bn), lambda i, j, k: (i, j)),
            pl.BlockSpec((1, 1, bm), lambda i, j, k: (i, 0, 0)),
        ],
        out_shape=[
            jax.ShapeDtypeStruct((m, n), F32),
            jax.ShapeDtypeStruct((m // bm, 1, bm), F32),
        ],
        scratch_shapes=[pltpu.VMEM((bm, bn), F32)],
    )(L, R)


def _score_body(h_ref, pw_ref, s_ref, hs_ref):
    pw = pw_ref[...]
    nrm = lax.rsqrt(jnp.sum(pw * pw))
    h = h_ref[...]
    t = jnp.sum(h * pw[None, :], axis=1) * nrm
    s = jnp.tanh(t)
    s_ref[...] = s
    hs_ref[...] = h * s[:, None]


def _score(h, pw, bm=512):
    n = h.shape[0]
    return pl.pallas_call(
        _score_body,
        grid=(n // bm,),
        in_specs=[
            pl.BlockSpec((bm, D), lambda i: (i, 0)),
            pl.BlockSpec((D,), lambda i: (0,)),
        ],
        out_specs=[
            pl.BlockSpec((bm,), lambda i: (i,)),
            pl.BlockSpec((bm, D), lambda i: (i, 0)),
        ],
        out_shape=[
            jax.ShapeDtypeStruct((n,), F32),
            jax.ShapeDtypeStruct((n, D), F32),
        ],
    )(h, pw)


def kernel(x, edge_index, Wd0, bd0, Wd1, bd1, Wd2, bd2, Wd3, bd3,
           pw0, pw1, pw2, Wu0, bu0, Wu1, bu1, Wu2, bu2):
    Wd = [Wd0, Wd1, Wd2, Wd3]
    bd = [bd0, bd1, bd2, bd3]
    pw = [pw0, pw1, pw2]
    Wu = [Wu0, Wu1, Wu2]
    bu = [bu0, bu1, bu2]

    u = edge_index[0]
    v = edge_index[1]
    ep = ((E0 + NW * 512 - 1) // (NW * 512)) * (NW * 512)
    u_p = jnp.concatenate([u, jnp.full((ep - E0,), NP0, jnp.int32)])
    v_p = jnp.concatenate([v, jnp.zeros((ep - E0,), jnp.int32)])
    u2 = jnp.stack([jnp.where(u_p < NPH, u_p, NPH),
                    jnp.where(u_p >= NPH, u_p - NPH, NPH)])

    xp = jnp.pad(x, ((0, NP0 - N0), (0, 0)))

    zb = jnp.zeros((D,), F32)
    deg0 = jnp.zeros((NP0,), F32).at[u].add(1.0) + 2.0
    Z0 = _zw(xp, Wd[0], deg0, zb)
    Sh = _spmm_sc(Z0, u2, v_p)
    S = jnp.concatenate([Sh[0], Sh[1]], axis=0)
    h = _combine(S, Z0, deg0, bd[0], coef=2.0, relu=True, nvalid=N0)
    xs = [h]
    degs = [deg0]
    ranks = []
    As = [None]
    A = None
    for i in range(1, DEPTH + 1):
        n_prev, np_prev = (N0, NP0) if i == 1 else LVL[i - 1]
        k, kp = LVL[i]
        score, hs = _score(h, pw[i - 1])
        _, perm = lax.top_k(score[:n_prev], k)
        rank = jnp.full((np_prev,), k, jnp.int32).at[perm].set(
            jnp.arange(k, dtype=jnp.int32))
        perm_p = jnp.concatenate([perm, jnp.zeros((kp - k,), jnp.int32)])
        if i == 1:
            ar = jnp.arange(k, dtype=jnp.int32)
            one = jnp.ones((ep + k,), jnp.bfloat16)
            lr = jnp.concatenate([rank[u_p], ar])
            lc = jnp.concatenate([v_p, perm])
            L = jnp.zeros((kp, np_prev), jnp.bfloat16).at[lr, lc].add(one)
            rr_ = jnp.concatenate([u_p, perm])
            rc = jnp.concatenate([rank[v_p], ar])
            R = jnp.zeros((np_prev, kp), jnp.bfloat16).at[rr_, rc].add(one)
        else:
            L = _gather_rows(A, perm_p, C=16 if i == 2 else 8)
            R = _mm_sel(A, _onehot(perm_p, A.shape[0]))
        B, rs = _bigmm(L, R, nvalid=k)
        deg = rs.reshape(-1) + 1.0
        hg = _gather_rows(hs, perm_p, C=min(160, kp // NW))
        Zl = _zw(hg, Wd[i], deg, zb)
        h = _dense_gcn(B, Zl, deg, bd[i], relu=True, nvalid=k)
        A = B
        ranks.append(rank)
        degs.append(deg)
        if i < DEPTH:
            xs.append(h)
            As.append(B)
    for i in range(DEPTH):
        j = DEPTH - 1 - i
        res = xs[j]
        up = _gather_rows(h, ranks[j], C=min(160, ranks[j].shape[0] // NW))
        if j == 0:
            Zu = _zw(res, Wu[i], degs[0], zb, add=up)
            Suh = _spmm_sc(Zu, u2, v_p)
            Su = jnp.concatenate([Suh[0], Suh[1]], axis=0)
            h = _combine(Su, Zu, degs[0], bu[i], coef=2.0,
                         relu=False, nvalid=N0)
        else:
            Zu = _zw(res, Wu[i], degs[j], zb, add=up)
            h = _dense_gcn(As[j], Zu, degs[j], bu[i], relu=(i < DEPTH - 1),
                           nvalid=LVL[j][0])
    return h[:N0]


# speedup vs baseline: 1.8891x; 1.8891x over previous
import functools
import math

import jax
import jax.numpy as jnp
from jax import lax
from jax.experimental import pallas as pl
from jax.experimental.pallas import tpu as pltpu
from jax.experimental.pallas import tpu_sc as plsc

N0 = 10000
NP0 = 10240
E0 = 320000
D = 128
DEPTH = 3
NW = 32
F32 = jnp.float32

LVL = {1: (5000, 5120), 2: (2500, 2560), 3: (1250, 1280)}


NPH = NP0 // 2
NACC = NPH + 128


def _spmm_sc_body(y_hbm, uu_hbm, vv_hbm, zeros_hbm, out_hbm,
                  ubuf, vbuf, rows, acc_sh, sem):
    C = ubuf.shape[0]
    cid = lax.axis_index("c")
    sid = lax.axis_index("s")
    rpt = NACC // 16
    pltpu.sync_copy(zeros_hbm.at[pl.ds(sid * rpt, rpt)],
                    acc_sh.at[pl.ds(sid * rpt, rpt)])
    plsc.subcore_barrier()
    ep_total = vv_hbm.shape[0]
    epw = ep_total // 16
    nch = epw // C

    def chunk(t, carry):
        off = sid * epw + t * C
        pltpu.sync_copy(uu_hbm.at[cid].at[pl.ds(off, C)], ubuf)
        pltpu.sync_copy(vv_hbm.at[pl.ds(off, C)], vbuf)
        pltpu.async_copy(y_hbm.at[vbuf], rows, sem).wait()
        pltpu.sync_copy(rows, acc_sh.at[ubuf], add=True)
        return carry

    lax.fori_loop(0, nch, chunk, 0)
    plsc.subcore_barrier()
    dpt = NPH // 16
    pltpu.sync_copy(acc_sh.at[pl.ds(sid * dpt, dpt)],
                    out_hbm.at[cid].at[pl.ds(sid * dpt, dpt)])


def _spmm_sc(y, uu2, vv):
    C = 512
    zeros = jnp.zeros((NACC, D), F32)
    mesh = plsc.VectorSubcoreMesh(core_axis_name="c", subcore_axis_name="s")
    f = pl.kernel(
        _spmm_sc_body,
        out_type=jax.ShapeDtypeStruct((2, NPH, D), F32),
        mesh=mesh,
        scratch_types=[
            pltpu.VMEM((C,), jnp.int32),
            pltpu.VMEM((C,), jnp.int32),
            pltpu.VMEM((C, D), F32),
            pltpu.VMEM_SHARED((NACC, D), F32),
            pltpu.SemaphoreType.DMA,
        ],
    )
    return f(y, uu2, vv, zeros)


def _gather_rows_body(tab_hbm, idx_hbm, out_hbm, ibuf, rows, sem, *, C):
    cid = lax.axis_index("c")
    sid = lax.axis_index("s")
    wid = sid * 2 + cid
    B = idx_hbm.shape[0]
    bpw = B // NW
    nch = bpw // C

    def chunk(t, carry):
        off = wid * bpw + t * C
        pltpu.sync_copy(idx_hbm.at[pl.ds(off, C)], ibuf)
        pltpu.async_copy(tab_hbm.at[ibuf], rows, sem).wait()
        pltpu.sync_copy(rows, out_hbm.at[pl.ds(off, C)])
        return carry

    lax.fori_loop(0, nch, chunk, 0)


def _gather_rows(tab, idx, C):
    B = idx.shape[0]
    W = tab.shape[1]
    mesh = plsc.VectorSubcoreMesh(core_axis_name="c", subcore_axis_name="s")
    f = pl.kernel(
        functools.partial(_gather_rows_body, C=C),
        out_type=jax.ShapeDtypeStruct((B, W), tab.dtype),
        mesh=mesh,
        scratch_types=[
            pltpu.VMEM((C,), jnp.int32),
            pltpu.VMEM((C, W), tab.dtype),
            pltpu.SemaphoreType.DMA,
        ],
    )
    return f(tab, idx)


def _onehot_body(perm_ref, o_ref, *, bm, bn):
    i = pl.program_id(0)
    j = pl.program_id(1)
    rows = i * bm + lax.broadcasted_iota(jnp.int32, (bm, bn), 0)
    pv = perm_ref[0, 0, :]
    o_ref[...] = (rows == pv[None, :]).astype(jnp.bfloat16)


def _onehot(perm_p, np_rows, bm=256, bn=256):
    kp = perm_p.shape[0]
    p3 = perm_p.reshape(1, 1, kp)
    return pl.pallas_call(
        functools.partial(_onehot_body, bm=bm, bn=bn),
        grid=(np_rows // bm, kp // bn),
        in_specs=[pl.BlockSpec((1, 1, bn), lambda i, j: (0, 0, j))],
        out_specs=pl.BlockSpec((bm, bn), lambda i, j: (i, j)),
        out_shape=jax.ShapeDtypeStruct((np_rows, kp), jnp.bfloat16),
    )(p3)


def _mm_body(a_ref, b_ref, s2_ref, o_ref, acc, *, nk, add_sel):
    kk = pl.program_id(2)

    @pl.when(kk == 0)
    def _():
        acc[...] = jnp.zeros_like(acc)

    a = a_ref[...].astype(jnp.bfloat16)
    acc[...] += jnp.dot(a, b_ref[...], preferred_element_type=F32)

    @pl.when(kk == nk - 1)
    def _():
        out = acc[...]
        if add_sel:
            out = out + s2_ref[...].astype(F32)
        o_ref[...] = out.astype(jnp.bfloat16)


def _mm_sel(A, S, add_sel=False, bm=1280, bn=1280, bk=512):
    m, K = A.shape
    n = S.shape[1]
    bm = min(bm, m)
    bn = min(bn, n)
    nk = K // bk
    return pl.pallas_call(
        functools.partial(_mm_body, nk=nk, add_sel=add_sel),
        grid=(m // bm, n // bn, nk),
        in_specs=[
            pl.BlockSpec((bm, bk), lambda i, j, k: (i, k)),
            pl.BlockSpec((bk, bn), lambda i, j, k: (k, j)),
            pl.BlockSpec((bm, bn), lambda i, j, k: (i, j)),
        ],
        out_specs=pl.BlockSpec((bm, bn), lambda i, j, k: (i, j)),
        out_shape=jax.ShapeDtypeStruct((m, n), jnp.bfloat16),
        scratch_shapes=[pltpu.VMEM((bm, bn), F32)],
    )(A, S, S)


def _rowsum_body(a_ref, o_ref, acc, *, nk, bm):
    kk = pl.program_id(1)

    @pl.when(kk == 0)
    def _():
        acc[...] = jnp.zeros_like(acc)

    acc[...] += jnp.sum(a_ref[...], axis=1).reshape(1, 1, bm)

    @pl.when(kk == nk - 1)
    def _():
        o_ref[...] = acc[...]


def _rowsum(A, bm=256, bk=2048):
    m, K = A.shape
    nk = K // bk
    out = pl.pallas_call(
        functools.partial(_rowsum_body, nk=nk, bm=bm),
        grid=(m // bm, nk),
        in_specs=[pl.BlockSpec((bm, bk), lambda i, k: (i, k))],
        out_specs=pl.BlockSpec((1, 1, bm), lambda i, k: (i, 0, 0)),
        out_shape=jax.ShapeDtypeStruct((m // bm, 1, bm), F32),
        scratch_shapes=[pltpu.VMEM((1, 1, bm), F32)],
    )(A)
    return out.reshape(m)


def _zw_body(x_ref, w_ref, deg_ref, b_ref, add_ref, o_ref, *, use_add, relu):
    x = x_ref[...]
    if use_add:
        x = x + add_ref[...]
    acc = jnp.dot(x, w_ref[...], preferred_element_type=F32)
    deg = deg_ref[...]
    rs = jnp.where(deg > 0.0, lax.rsqrt(deg), 0.0)
    out = acc * rs[:, None] + b_ref[...][None, :]
    if relu:
        out = jnp.maximum(out, 0.0)
    o_ref[...] = out


def _zw(x, w, deg, b, add=None, relu=False, bm=256):
    n = x.shape[0]
    use_add = add is not None
    if add is None:
        add = x
    grid = (n // bm,)
    return pl.pallas_call(
        functools.partial(_zw_body, use_add=use_add, relu=relu),
        grid=grid,
        in_specs=[
            pl.BlockSpec((bm, D), lambda i: (i, 0)),
            pl.BlockSpec((D, D), lambda i: (0, 0)),
            pl.BlockSpec((bm,), lambda i: (i,)),
            pl.BlockSpec((D,), lambda i: (0,)),
            pl.BlockSpec((bm, D), lambda i: (i, 0)),
        ],
        out_specs=pl.BlockSpec((bm, D), lambda i: (i, 0)),
        out_shape=jax.ShapeDtypeStruct((n, D), F32),
    )(x, w, deg, b, add)


def _combine_body(s0_ref, z_ref, deg_ref, b_ref, o_ref,
                  *, coef, relu, nvalid, bm):
    i = pl.program_id(0)
    s = s0_ref[...] + coef * z_ref[...]
    deg = deg_ref[...]
    rs = jnp.where(deg > 0.0, lax.rsqrt(deg), 0.0)
    out = s * rs[:, None] + b_ref[...][None, :]
    if relu:
        out = jnp.maximum(out, 0.0)
    rows = i * bm + lax.broadcasted_iota(jnp.int32, (bm, 1), 0)
    o_ref[...] = jnp.where(rows < nvalid, out, 0.0)


def _combine(s0, z, deg, b, coef, relu, nvalid, bm=512):
    n = z.shape[0]
    return pl.pallas_call(
        functools.partial(_combine_body, coef=coef, relu=relu,
                          nvalid=nvalid, bm=bm),
        grid=(n // bm,),
        in_specs=[
            pl.BlockSpec((bm, D), lambda i: (i, 0)),
            pl.BlockSpec((bm, D), lambda i: (i, 0)),
            pl.BlockSpec((bm,), lambda i: (i,)),
            pl.BlockSpec((D,), lambda i: (0,)),
        ],
        out_specs=pl.BlockSpec((bm, D), lambda i: (i, 0)),
        out_shape=jax.ShapeDtypeStruct((n, D), F32),
    )(s0, z, deg, b)


def _dense_gcn_body(m_ref, z_ref, zrow_ref, deg_ref, b_ref, o_ref, acc,
                    *, nk, relu, nvalid, bm):
    kk = pl.program_id(1)

    @pl.when(kk == 0)
    def _():
        acc[...] = jnp.zeros_like(acc)

    acc[...] += jnp.dot(m_ref[...], z_ref[...], preferred_element_type=F32)
    i = pl.program_id(0)

    @pl.when(kk == nk - 1)
    def _():
        deg = deg_ref[...]
        rs = jnp.where(deg > 0.0, lax.rsqrt(deg), 0.0)
        out = rs[:, None] * (acc[...] + zrow_ref[...]) + b_ref[...][None, :]
        if relu:
            out = jnp.maximum(out, 0.0)
        rows = i * bm + lax.broadcasted_iota(jnp.int32, (bm, 1), 0)
        o_ref[...] = jnp.where(rows < nvalid, out, 0.0)


def _dense_gcn(M, Z, deg, b, relu, nvalid, bm=256, bk=640):
    n = M.shape[0]
    nk = n // bk
    return pl.pallas_call(
        functools.partial(_dense_gcn_body, nk=nk, relu=relu,
                          nvalid=nvalid, bm=bm),
        grid=(n // bm, nk),
        in_specs=[
            pl.BlockSpec((bm, bk), lambda i, k: (i, k)),
            pl.BlockSpec((bk, D), lambda i, k: (k, 0)),
            pl.BlockSpec((bm, D), lambda i, k: (i, 0)),
            pl.BlockSpec((bm,), lambda i, k: (i,)),
            pl.BlockSpec((D,), lambda i, k: (0,)),
        ],
        out_specs=pl.BlockSpec((bm, D), lambda i, k: (i, 0)),
        out_shape=jax.ShapeDtypeStruct((n, D), F32),
        scratch_shapes=[pltpu.VMEM((bm, D), F32)],
    )(M, Z, Z, deg, b)


def _bigmm_body(l_ref, r_ref, radd_ref, g_ref, d_ref, acc,
                *, nk, nvalid, bm, bn, use_radd):
    i = pl.program_id(0)
    j = pl.program_id(1)
    kk = pl.program_id(2)

    @pl.when(kk == 0)
    def _():
        acc[...] = jnp.zeros_like(acc)

    a = l_ref[...].astype(jnp.bfloat16)
    b = r_ref[...].astype(jnp.bfloat16)
    acc[...] += jnp.dot(a, b, preferred_element_type=F32)

    @pl.when(kk == nk - 1)
    def _():
        rows = i * bm + lax.broadcasted_iota(jnp.int32, (bm, bn), 0)
        cols = j * bn + lax.broadcasted_iota(jnp.int32, (bm, bn), 1)
        valid = (rows < nvalid) & (cols < nvalid)
        full = acc[...]
        if use_radd:
            full = full + radd_ref[...].astype(F32)
        g = jnp.where(rows == cols, 1.0, full)
        g = jnp.where(valid, g, 0.0)
        g_ref[...] = g
        part = jnp.sum(g, axis=1).reshape(1, 1, bm)

        @pl.when(j == 0)
        def _():
            d_ref[...] = part

        @pl.when(j > 0)
        def _():
            d_ref[...] += part


def _bigmm(L, R, nvalid, radd=None, bm=1280, bn=1280, bk=512):
    m, K = L.shape
    n = R.shape[1]
    bm = min(bm, m)
    bn = min(bn, n)
    nk = K // bk
    use_radd = radd is not None
    if radd is None:
        radd = jnp.zeros((m, n), jnp.bfloat16)
    return pl.pallas_call(
        functools.partial(_bigmm_body, nk=nk, nvalid=nvalid,
                          bm=bm, bn=bn, use_radd=use_radd),
        grid=(m // bm, n // bn, nk),
        in_specs=[
            pl.BlockSpec((bm, bk), lambda i, j, k: (i, k)),
            pl.BlockSpec((bk, bn), lambda i, j, k: (k, j)),
            pl.BlockSpec((bm, bn), lambda i, j, k: (i, j)),
        ],
        out_specs=[
            pl.BlockSpec((bm, bn), lambda i, j, k: (i, j)),
            pl.BlockSpec((1, 1, bm), lambda i, j, k: (i, 0, 0)),
        ],
        out_shape=[
            jax.ShapeDtypeStruct((m, n), F32),
            jax.ShapeDtypeStruct((m // bm, 1, bm), F32),
        ],
        scratch_shapes=[pltpu.VMEM((bm, bn), F32)],
    )(L, R, radd)


def _score_body(h_ref, pw_ref, s_ref, hs_ref):
    pw = pw_ref[...]
    nrm = lax.rsqrt(jnp.sum(pw * pw))
    h = h_ref[...]
    t = jnp.sum(h * pw[None, :], axis=1) * nrm
    s = jnp.tanh(t)
    s_ref[...] = s
    hs_ref[...] = h * s[:, None]


def _score(h, pw, bm=512):
    n = h.shape[0]
    return pl.pallas_call(
        _score_body,
        grid=(n // bm,),
        in_specs=[
            pl.BlockSpec((bm, D), lambda i: (i, 0)),
            pl.BlockSpec((D,), lambda i: (0,)),
        ],
        out_specs=[
            pl.BlockSpec((bm,), lambda i: (i,)),
            pl.BlockSpec((bm, D), lambda i: (i, 0)),
        ],
        out_shape=[
            jax.ShapeDtypeStruct((n,), F32),
            jax.ShapeDtypeStruct((n, D), F32),
        ],
    )(h, pw)


def kernel(x, edge_index, Wd0, bd0, Wd1, bd1, Wd2, bd2, Wd3, bd3,
           pw0, pw1, pw2, Wu0, bu0, Wu1, bu1, Wu2, bu2):
    Wd = [Wd0, Wd1, Wd2, Wd3]
    bd = [bd0, bd1, bd2, bd3]
    pw = [pw0, pw1, pw2]
    Wu = [Wu0, Wu1, Wu2]
    bu = [bu0, bu1, bu2]

    u = edge_index[0]
    v = edge_index[1]
    ep = ((E0 + NW * 512 - 1) // (NW * 512)) * (NW * 512)
    u_p = jnp.concatenate([u, jnp.full((ep - E0,), NP0, jnp.int32)])
    v_p = jnp.concatenate([v, jnp.zeros((ep - E0,), jnp.int32)])
    u2 = jnp.stack([jnp.where(u_p < NPH, u_p, NPH),
                    jnp.where(u_p >= NPH, u_p - NPH, NPH)])

    xp = jnp.pad(x, ((0, NP0 - N0), (0, 0)))

    zb = jnp.zeros((D,), F32)
    Afull = jnp.zeros((NP0, NP0), F32).at[u, v].add(1.0)
    deg0 = _rowsum(Afull) + 2.0
    Z0 = _zw(xp, Wd[0], deg0, zb)
    Sh = _spmm_sc(Z0, u2, v_p)
    S = jnp.concatenate([Sh[0], Sh[1]], axis=0)
    h = _combine(S, Z0, deg0, bd[0], coef=2.0, relu=True, nvalid=N0)
    xs = [h]
    degs = [deg0]
    ranks = []
    As = [None]
    A = None
    for i in range(1, DEPTH + 1):
        n_prev, np_prev = (N0, NP0) if i == 1 else LVL[i - 1]
        k, kp = LVL[i]
        score, hs = _score(h, pw[i - 1])
        _, perm = lax.top_k(score[:n_prev], k)
        rank = jnp.full((np_prev,), k, jnp.int32).at[perm].set(
            jnp.arange(k, dtype=jnp.int32))
        perm_p = jnp.concatenate([perm, jnp.zeros((kp - k,), jnp.int32)])
        if i == 1:
            R = _mm_sel(Afull, _onehot(perm_p, np_prev), add_sel=True)
            L = _gather_rows(Afull, perm_p, C=8)
            Rpk = lax.bitcast_convert_type(R.reshape(np_prev, kp // 2, 2), F32)
            Rgp = _gather_rows(Rpk, perm_p, C=16)
            Rg = lax.bitcast_convert_type(Rgp, jnp.bfloat16).reshape(kp, kp)
            B, rs = _bigmm(L, R, nvalid=k, radd=Rg)
        else:
            L = _gather_rows(A, perm_p, C=16 if i == 2 else 8)
            R = _mm_sel(A, _onehot(perm_p, A.shape[0]))
            B, rs = _bigmm(L, R, nvalid=k)
        deg = rs.reshape(-1) + 1.0
        hg = _gather_rows(hs, perm_p, C=min(160, kp // NW))
        Zl = _zw(hg, Wd[i], deg, zb)
        h = _dense_gcn(B, Zl, deg, bd[i], relu=True, nvalid=k)
        A = B
        ranks.append(rank)
        degs.append(deg)
        if i < DEPTH:
            xs.append(h)
            As.append(B)
    for i in range(DEPTH):
        j = DEPTH - 1 - i
        res = xs[j]
        up = _gather_rows(h, ranks[j], C=min(160, ranks[j].shape[0] // NW))
        if j == 0:
            Zu = _zw(res, Wu[i], degs[0], zb, add=up)
            Suh = _spmm_sc(Zu, u2, v_p)
            Su = jnp.concatenate([Suh[0], Suh[1]], axis=0)
            h = _combine(Su, Zu, degs[0], bu[i], coef=2.0,
                         relu=False, nvalid=N0)
        else:
            Zu = _zw(res, Wu[i], degs[j], zb, add=up)
            h = _dense_gcn(As[j], Zu, degs[j], bu[i], relu=(i < DEPTH - 1),
                           nvalid=LVL[j][0])
    return h[:N0]
